# Initial kernel scaffold; baseline (speedup 1.0000x reference)
import jax
import jax.numpy as jnp
from jax.experimental import pallas as pl


def kernel(tokens, cls_token):
    raise NotImplementedError("write your pallas kernel here")


# speedup vs baseline: 11.4403x; 11.4403x over previous
import functools

import jax
import jax.numpy as jnp
from jax.experimental import pallas as pl

_MERGE_RATIO = 0.3
_MIN_TOKENS = 4


def _l2norm(x, eps=1e-12):
    n = jnp.linalg.norm(x, axis=-1, keepdims=True)
    return x / jnp.maximum(n, eps)


def _pow2_neg(k):
    e = jnp.maximum(127 - k, 0) << 23
    return jax.lax.bitcast_convert_type(e.astype(jnp.int32), jnp.float32)


def _merge_kernel(tokens_ref, sn_ref, dn_ref, cls_ref,
                  merged_ref, midx_ref, ind_ref, *, split, r):
    f32 = jnp.float32
    hi = jax.lax.Precision.HIGHEST
    src = tokens_ref[0, :split, :]
    dst = tokens_ref[0, split:, :]

    sim = jax.lax.dot_general(
        sn_ref[0], dn_ref[0], (((1,), (1,)), ((), ())),
        preferred_element_type=f32)

    rows = jax.lax.broadcasted_iota(jnp.int32, (split, split), 0)
    cols = jax.lax.broadcasted_iota(jnp.int32, (split, split), 1)
    rowsf = rows.astype(f32)
    colsf = cols.astype(f32)

    scores = jnp.max(sim, axis=1, keepdims=True)
    ind = jnp.min(jnp.where(sim == scores, colsf, float(split)),
                  axis=1, keepdims=True)

    scores_r = jnp.transpose(scores)
    ind_r = jnp.transpose(ind)

    rank_cond = (scores_r > scores) | ((scores_r == scores) & (colsf < rowsf))
    rank = jnp.sum(rank_cond, axis=1, keepdims=True).astype(f32)
    rank_r = jnp.transpose(rank)

    merged_col = (rank < r).astype(f32)
    merged_r = jnp.transpose(merged_col)
    keep_r = 1.0 - merged_r

    same_dst = (ind_r == ind) & (merged_col > 0) & (merged_r > 0)
    later = rank_r > rank
    c = jnp.sum(same_dst & later, axis=1, keepdims=True)
    w = _pow2_neg(c + 1) * merged_col
    w_r = jnp.transpose(w)

    hit = (ind_r == rowsf).astype(f32)
    A = hit * w_r
    S_add = jax.lax.dot_general(
        A, src, (((1,), (0,)), ((), ())), preferred_element_type=f32,
        precision=hi)
    m = jnp.sum(hit * merged_r, axis=1, keepdims=True)
    dst_m = dst * _pow2_neg(m.astype(jnp.int32)) + S_add

    kept_rank = jnp.sum(keep_r * (colsf < rowsf).astype(f32),
                        axis=1, keepdims=True)
    kr_r = jnp.transpose(kept_rank)
    G = (kr_r == rowsf).astype(f32) * keep_r
    kept_full = jax.lax.dot_general(
        G, src, (((1,), (0,)), ((), ())), preferred_element_type=f32,
        precision=hi)

    merged_ref[0] = jnp.concatenate(
        [cls_ref[0], kept_full[: split - r], dst_m], axis=0)

    sel = (rank_r == rowsf).astype(f32)
    midx_col = jnp.sum(sel * colsf, axis=1, keepdims=True)
    midx_ref[0] = jnp.transpose(midx_col)[:, :r].astype(jnp.int32)
    ind_ref[0] = ind_r.astype(jnp.int32)


def kernel(tokens, cls_token):
    B, N, D = tokens.shape
    split = N // 2
    r = int(N * _MERGE_RATIO)
    r = min(r, N - _MIN_TOKENS)
    kept = split - r
    n_out = 1 + kept + split

    src_n = _l2norm(tokens[:, :split])
    dst_n = _l2norm(tokens[:, split:])

    kfn = functools.partial(_merge_kernel, split=split, r=r)
    merged, midx, ind = pl.pallas_call(
        kfn,
        grid=(B,),
        in_specs=[
            pl.BlockSpec((1, N, D), lambda b: (b, 0, 0)),
            pl.BlockSpec((1, split, D), lambda b: (b, 0, 0)),
            pl.BlockSpec((1, split, D), lambda b: (b, 0, 0)),
            pl.BlockSpec((1, 1, D), lambda b: (b, 0, 0)),
        ],
        out_specs=[
            pl.BlockSpec((1, n_out, D), lambda b: (b, 0, 0)),
            pl.BlockSpec((1, 1, r), lambda b: (b, 0, 0)),
            pl.BlockSpec((1, 1, split), lambda b: (b, 0, 0)),
        ],
        out_shape=[
            jax.ShapeDtypeStruct((B, n_out, D), tokens.dtype),
            jax.ShapeDtypeStruct((B, 1, r), jnp.int32),
            jax.ShapeDtypeStruct((B, 1, split), jnp.int32),
        ],
    )(tokens, src_n, dst_n, cls_token)
    return merged, midx.reshape(B, r), ind.reshape(B, split)


# speedup vs baseline: 14.2457x; 1.2452x over previous
import functools

import jax
import jax.numpy as jnp
from jax.experimental import pallas as pl

_MERGE_RATIO = 0.3
_MIN_TOKENS = 4


def _pow2_neg(k):
    e = jnp.maximum(127 - k, 0) << 23
    return jax.lax.bitcast_convert_type(e.astype(jnp.int32), jnp.float32)


def _merge_kernel(sn_ref, dn_ref, ns_ref, nd_ref, cls_ref,
                  merged_ref, midx_ref, ind_ref, *, split, r, rows_c):
    f32 = jnp.float32
    kept = split - r
    sn = sn_ref[0]
    dn = dn_ref[0]
    src = sn * jnp.transpose(ns_ref[0])
    dst = dn * jnp.transpose(nd_ref[0])

    sim = jax.lax.dot_general(
        sn, dn, (((1,), (1,)), ((), ())), preferred_element_type=f32)

    rowsf = jax.lax.broadcasted_iota(jnp.int32, (split, split), 0).astype(f32)
    colsf = jax.lax.broadcasted_iota(jnp.int32, (split, split), 1).astype(f32)
    ones_col = jnp.ones((split, 1), f32)

    def rowsum(mat):
        return jax.lax.dot_general(
            mat, ones_col, (((1,), (0,)), ((), ())),
            preferred_element_type=f32)

    scores = jnp.max(sim, axis=1, keepdims=True)
    ind = jnp.min(jnp.where(sim == scores, colsf, float(split)),
                  axis=1, keepdims=True)

    scores_r = jnp.transpose(scores)
    ind_r = jnp.transpose(ind)

    lower = colsf < rowsf
    rank_cond = (scores_r > scores) | ((scores_r == scores) & lower)
    rank = rowsum(rank_cond.astype(f32))
    rank_r = jnp.transpose(rank)

    merged_col = (rank < r).astype(f32)
    merged_r = jnp.transpose(merged_col)

    same_dst = (ind_r == ind) & (merged_col > 0) & (merged_r > 0)
    later = rank_r > rank
    c = rowsum((same_dst & later).astype(f32))
    w = _pow2_neg((c + 1.0).astype(jnp.int32)) * merged_col
    w_r = jnp.transpose(w)

    kept_rank = jax.lax.dot_general(
        lower.astype(f32), 1.0 - merged_col, (((1,), (0,)), ((), ())),
        preferred_element_type=f32)
    kr_r = jnp.transpose(kept_rank)

    t_r = jnp.where(merged_r > 0, float(kept) + ind_r, kr_r)
    v_r = jnp.where(merged_r > 0, w_r, 1.0)
    q_iota = jax.lax.broadcasted_iota(
        jnp.int32, (rows_c, split), 0).astype(f32)
    E = (q_iota == t_r).astype(f32)
    C = E * v_r
    out2 = jax.lax.dot_general(
        C, src, (((1,), (0,)), ((), ())), preferred_element_type=f32)
    m = jax.lax.dot_general(
        E, merged_col, (((1,), (0,)), ((), ())),
        preferred_element_type=f32)[kept:kept + split]

    dst_m = dst * _pow2_neg(m.astype(jnp.int32)) + out2[kept:kept + split]
    merged_ref[0] = jnp.concatenate(
        [cls_ref[0], out2[:kept], dst_m], axis=0)

    sel = (rank_r == rowsf).astype(f32)
    midx_col = jnp.sum(sel * colsf, axis=1, keepdims=True)
    midx_ref[0] = jnp.transpose(midx_col)[:, :r].astype(jnp.int32)
    ind_ref[0] = ind_r.astype(jnp.int32)


def kernel(tokens, cls_token):
    B, N, D = tokens.shape
    split = N // 2
    r = int(N * _MERGE_RATIO)
    r = min(r, N - _MIN_TOKENS)
    kept = split - r
    n_out = 1 + kept + split
    rows_c = ((kept + split + 7) // 8) * 8

    src, dst = tokens[:, :split], tokens[:, split:]
    ns = jnp.linalg.norm(src, axis=-1, keepdims=True)
    nd = jnp.linalg.norm(dst, axis=-1, keepdims=True)
    ns = jnp.maximum(ns, 1e-12)
    nd = jnp.maximum(nd, 1e-12)
    src_n = src / ns
    dst_n = dst / nd

    kfn = functools.partial(_merge_kernel, split=split, r=r, rows_c=rows_c)
    merged, midx, ind = pl.pallas_call(
        kfn,
        grid=(B,),
        in_specs=[
            pl.BlockSpec((1, split, D), lambda b: (b, 0, 0)),
            pl.BlockSpec((1, split, D), lambda b: (b, 0, 0)),
            pl.BlockSpec((1, 1, split), lambda b: (b, 0, 0)),
            pl.BlockSpec((1, 1, split), lambda b: (b, 0, 0)),
            pl.BlockSpec((1, 1, D), lambda b: (b, 0, 0)),
        ],
        out_specs=[
            pl.BlockSpec((1, n_out, D), lambda b: (b, 0, 0)),
            pl.BlockSpec((1, 1, r), lambda b: (b, 0, 0)),
            pl.BlockSpec((1, 1, split), lambda b: (b, 0, 0)),
        ],
        out_shape=[
            jax.ShapeDtypeStruct((B, n_out, D), tokens.dtype),
            jax.ShapeDtypeStruct((B, 1, r), jnp.int32),
            jax.ShapeDtypeStruct((B, 1, split), jnp.int32),
        ],
    )(src_n, dst_n, ns.reshape(B, 1, split), nd.reshape(B, 1, split),
      cls_token)
    return merged, midx.reshape(B, r), ind.reshape(B, split)


# speedup vs baseline: 14.9414x; 1.0488x over previous
import functools

import jax
import jax.numpy as jnp
from jax.experimental import pallas as pl

_MERGE_RATIO = 0.3
_MIN_TOKENS = 4


def _pow2_neg(k):
    e = jnp.maximum(127 - k, 0) << 23
    return jax.lax.bitcast_convert_type(e.astype(jnp.int32), jnp.float32)


def _merge_kernel(sn_ref, dn_ref, ns_ref, nd_ref, cls_ref,
                  merged_ref, midx_ref, ind_ref, *, split, r, rows_c, bb):
    for k in range(bb):
        _merge_one(sn_ref, dn_ref, ns_ref, nd_ref, cls_ref,
                   merged_ref, midx_ref, ind_ref, k,
                   split=split, r=r, rows_c=rows_c)


def _merge_one(sn_ref, dn_ref, ns_ref, nd_ref, cls_ref,
               merged_ref, midx_ref, ind_ref, k, *, split, r, rows_c):
    f32 = jnp.float32
    kept = split - r
    sn = sn_ref[k]
    dn = dn_ref[k]
    src = sn * jnp.transpose(ns_ref[k])
    dst = dn * jnp.transpose(nd_ref[k])

    sim = jax.lax.dot_general(
        sn, dn, (((1,), (1,)), ((), ())), preferred_element_type=f32)

    rowsf = jax.lax.broadcasted_iota(jnp.int32, (split, split), 0).astype(f32)
    colsf = jax.lax.broadcasted_iota(jnp.int32, (split, split), 1).astype(f32)
    ones_col = jnp.ones((split, 1), f32)

    def rowsum(mat):
        return jax.lax.dot_general(
            mat, ones_col, (((1,), (0,)), ((), ())),
            preferred_element_type=f32)

    scores = jnp.max(sim, axis=1, keepdims=True)
    ind = jnp.min(jnp.where(sim == scores, colsf, float(split)),
                  axis=1, keepdims=True)

    scores_r = jnp.transpose(scores)
    ind_r = jnp.transpose(ind)

    lower = colsf < rowsf
    rank_cond = (scores_r > scores) | ((scores_r == scores) & lower)
    rank = rowsum(rank_cond.astype(f32))
    rank_r = jnp.transpose(rank)

    merged_col = (rank < r).astype(f32)
    merged_r = jnp.transpose(merged_col)

    same_dst = (ind_r == ind) & (merged_col > 0) & (merged_r > 0)
    later = rank_r > rank
    c = rowsum((same_dst & later).astype(f32))
    w = _pow2_neg((c + 1.0).astype(jnp.int32)) * merged_col
    w_r = jnp.transpose(w)

    kept_rank = jax.lax.dot_general(
        lower.astype(f32), 1.0 - merged_col, (((1,), (0,)), ((), ())),
        preferred_element_type=f32)
    kr_r = jnp.transpose(kept_rank)

    t_r = jnp.where(merged_r > 0, float(kept) + ind_r, kr_r)
    v_r = jnp.where(merged_r > 0, w_r, 1.0)
    q_iota = jax.lax.broadcasted_iota(
        jnp.int32, (rows_c, split), 0).astype(f32)
    E = (q_iota == t_r).astype(f32)
    C = E * v_r
    out2 = jax.lax.dot_general(
        C, src, (((1,), (0,)), ((), ())), preferred_element_type=f32)
    m = jax.lax.dot_general(
        E, merged_col, (((1,), (0,)), ((), ())),
        preferred_element_type=f32)[kept:kept + split]

    dst_m = dst * _pow2_neg(m.astype(jnp.int32)) + out2[kept:kept + split]
    merged_ref[k] = jnp.concatenate(
        [cls_ref[k], out2[:kept], dst_m], axis=0)

    sel = (rank_r == rowsf).astype(f32)
    midx_col = jnp.sum(sel * colsf, axis=1, keepdims=True)
    midx_ref[k] = jnp.transpose(midx_col)[:, :r].astype(jnp.int32)
    ind_ref[k] = ind_r.astype(jnp.int32)


def kernel(tokens, cls_token):
    B, N, D = tokens.shape
    split = N // 2
    r = int(N * _MERGE_RATIO)
    r = min(r, N - _MIN_TOKENS)
    kept = split - r
    n_out = 1 + kept + split
    rows_c = ((kept + split + 7) // 8) * 8

    src, dst = tokens[:, :split], tokens[:, split:]
    ns = jnp.linalg.norm(src, axis=-1, keepdims=True)
    nd = jnp.linalg.norm(dst, axis=-1, keepdims=True)
    ns = jnp.maximum(ns, 1e-12)
    nd = jnp.maximum(nd, 1e-12)
    src_n = src / ns
    dst_n = dst / nd

    bb = 2 if B % 2 == 0 else 1
    kfn = functools.partial(_merge_kernel, split=split, r=r, rows_c=rows_c,
                            bb=bb)
    merged, midx, ind = pl.pallas_call(
        kfn,
        grid=(B // bb,),
        in_specs=[
            pl.BlockSpec((bb, split, D), lambda b: (b, 0, 0)),
            pl.BlockSpec((bb, split, D), lambda b: (b, 0, 0)),
            pl.BlockSpec((bb, 1, split), lambda b: (b, 0, 0)),
            pl.BlockSpec((bb, 1, split), lambda b: (b, 0, 0)),
            pl.BlockSpec((bb, 1, D), lambda b: (b, 0, 0)),
        ],
        out_specs=[
            pl.BlockSpec((bb, n_out, D), lambda b: (b, 0, 0)),
            pl.BlockSpec((bb, 1, r), lambda b: (b, 0, 0)),
            pl.BlockSpec((bb, 1, split), lambda b: (b, 0, 0)),
        ],
        out_shape=[
            jax.ShapeDtypeStruct((B, n_out, D), tokens.dtype),
            jax.ShapeDtypeStruct((B, 1, r), jnp.int32),
            jax.ShapeDtypeStruct((B, 1, split), jnp.int32),
        ],
    )(src_n, dst_n, ns.reshape(B, 1, split), nd.reshape(B, 1, split),
      cls_token)
    return merged, midx.reshape(B, r), ind.reshape(B, split)


# speedup vs baseline: 15.0948x; 1.0103x over previous
import functools

import jax
import jax.numpy as jnp
from jax.experimental import pallas as pl

_MERGE_RATIO = 0.3
_MIN_TOKENS = 4


def _pow2_neg(k):
    e = jnp.maximum(127 - k, 0) << 23
    return jax.lax.bitcast_convert_type(e.astype(jnp.int32), jnp.float32)


def _merge_kernel(sn_ref, dn_ref, ns_ref, nd_ref, cls_ref,
                  merged_ref, midx_ref, ind_ref, *, split, r, rows_c, bb):
    for k in range(bb):
        _merge_one(sn_ref, dn_ref, ns_ref, nd_ref, cls_ref,
                   merged_ref, midx_ref, ind_ref, k,
                   split=split, r=r, rows_c=rows_c)


def _merge_one(sn_ref, dn_ref, ns_ref, nd_ref, cls_ref,
               merged_ref, midx_ref, ind_ref, k, *, split, r, rows_c):
    f32 = jnp.float32
    kept = split - r
    sn = sn_ref[k]
    dn = dn_ref[k]
    src = sn * jnp.transpose(ns_ref[k])
    dst = dn * jnp.transpose(nd_ref[k])

    sim = jax.lax.dot_general(
        sn, dn, (((1,), (1,)), ((), ())), preferred_element_type=f32)

    rowsf = jax.lax.broadcasted_iota(jnp.int32, (split, split), 0).astype(f32)
    colsf = jax.lax.broadcasted_iota(jnp.int32, (split, split), 1).astype(f32)
    ones_col = jnp.ones((split, 1), f32)

    def rowsum(mat):
        return jax.lax.dot_general(
            mat, ones_col, (((1,), (0,)), ((), ())),
            preferred_element_type=f32)

    scores = jnp.max(sim, axis=1, keepdims=True)
    ind = jnp.min(jnp.where(sim == scores, colsf, float(split)),
                  axis=1, keepdims=True)

    scores_r = jnp.transpose(scores)
    ind_r = jnp.transpose(ind)

    lower = colsf < rowsf
    rank_cond = (scores_r > scores) | ((scores_r == scores) & lower)
    rank = rowsum(rank_cond.astype(f32))
    rank_r = jnp.transpose(rank)

    merged_col = (rank < r).astype(f32)
    merged_r = jnp.transpose(merged_col)

    same_dst = (ind_r == ind) & (merged_col > 0) & (merged_r > 0)
    later = rank_r > rank
    c = rowsum((same_dst & later).astype(f32))
    w = _pow2_neg((c + 1.0).astype(jnp.int32)) * merged_col
    w_r = jnp.transpose(w)

    kept_rank = jax.lax.dot_general(
        lower.astype(f32), 1.0 - merged_col, (((1,), (0,)), ((), ())),
        preferred_element_type=f32)
    kr_r = jnp.transpose(kept_rank)

    t_r = jnp.where(merged_r > 0, float(kept) + ind_r, kr_r)
    v_r = jnp.where(merged_r > 0, w_r, 1.0)
    q_iota = jax.lax.broadcasted_iota(
        jnp.int32, (rows_c, split), 0).astype(f32)
    E = (q_iota == t_r).astype(f32)
    C = E * v_r
    out2 = jax.lax.dot_general(
        C, src, (((1,), (0,)), ((), ())), preferred_element_type=f32)
    m = jax.lax.dot_general(
        E, merged_col, (((1,), (0,)), ((), ())),
        preferred_element_type=f32)[kept:kept + split]

    dst_m = dst * _pow2_neg(m.astype(jnp.int32)) + out2[kept:kept + split]
    merged_ref[k] = jnp.concatenate(
        [cls_ref[k], out2[:kept], dst_m], axis=0)

    sel = (rank_r == rowsf).astype(f32)
    midx_col = jnp.sum(sel * colsf, axis=1, keepdims=True)
    midx_ref[k] = jnp.transpose(midx_col)[:, :r].astype(jnp.int32)
    ind_ref[k] = ind_r.astype(jnp.int32)


def kernel(tokens, cls_token):
    B, N, D = tokens.shape
    split = N // 2
    r = int(N * _MERGE_RATIO)
    r = min(r, N - _MIN_TOKENS)
    kept = split - r
    n_out = 1 + kept + split
    rows_c = ((kept + split + 7) // 8) * 8

    src, dst = tokens[:, :split], tokens[:, split:]
    ns = jnp.linalg.norm(src, axis=-1, keepdims=True)
    nd = jnp.linalg.norm(dst, axis=-1, keepdims=True)
    ns = jnp.maximum(ns, 1e-12)
    nd = jnp.maximum(nd, 1e-12)
    src_n = src / ns
    dst_n = dst / nd

    bb = 4 if B % 4 == 0 else 1
    kfn = functools.partial(_merge_kernel, split=split, r=r, rows_c=rows_c,
                            bb=bb)
    merged, midx, ind = pl.pallas_call(
        kfn,
        grid=(B // bb,),
        in_specs=[
            pl.BlockSpec((bb, split, D), lambda b: (b, 0, 0)),
            pl.BlockSpec((bb, split, D), lambda b: (b, 0, 0)),
            pl.BlockSpec((bb, 1, split), lambda b: (b, 0, 0)),
            pl.BlockSpec((bb, 1, split), lambda b: (b, 0, 0)),
            pl.BlockSpec((bb, 1, D), lambda b: (b, 0, 0)),
        ],
        out_specs=[
            pl.BlockSpec((bb, n_out, D), lambda b: (b, 0, 0)),
            pl.BlockSpec((bb, 1, r), lambda b: (b, 0, 0)),
            pl.BlockSpec((bb, 1, split), lambda b: (b, 0, 0)),
        ],
        out_shape=[
            jax.ShapeDtypeStruct((B, n_out, D), tokens.dtype),
            jax.ShapeDtypeStruct((B, 1, r), jnp.int32),
            jax.ShapeDtypeStruct((B, 1, split), jnp.int32),
        ],
    )(src_n, dst_n, ns.reshape(B, 1, split), nd.reshape(B, 1, split),
      cls_token)
    return merged, midx.reshape(B, r), ind.reshape(B, split)


# speedup vs baseline: 15.7423x; 1.0429x over previous
import functools

import jax
import jax.numpy as jnp
from jax.experimental import pallas as pl

_MERGE_RATIO = 0.3
_MIN_TOKENS = 4
_BIG = 8388608.0


def _pow2_neg(k):
    e = jnp.maximum(127 - k, 0) << 23
    return jax.lax.bitcast_convert_type(e.astype(jnp.int32), jnp.float32)


def _merge_kernel(sn_ref, dn_ref, ns_ref, nd_ref, cls_ref,
                  merged_ref, midx_ref, ind_ref, *, split, r, rows_c, bb):
    for k in range(bb):
        _merge_one(sn_ref, dn_ref, ns_ref, nd_ref, cls_ref,
                   merged_ref, midx_ref, ind_ref, k,
                   split=split, r=r, rows_c=rows_c)


def _merge_one(sn_ref, dn_ref, ns_ref, nd_ref, cls_ref,
               merged_ref, midx_ref, ind_ref, k, *, split, r, rows_c):
    f32 = jnp.float32
    kept = split - r
    sn = sn_ref[k]
    dn = dn_ref[k]
    ns_r = ns_ref[k]
    nd_c = jnp.transpose(nd_ref[k])

    sim = jax.lax.dot_general(
        sn, dn, (((1,), (1,)), ((), ())), preferred_element_type=f32)

    rowsf = jax.lax.broadcasted_iota(jnp.int32, (split, split), 0).astype(f32)
    colsf = jax.lax.broadcasted_iota(jnp.int32, (split, split), 1).astype(f32)
    ones_col = jnp.ones((split, 1), f32)

    scores = jnp.max(sim, axis=1, keepdims=True)
    ind = jnp.min(jnp.where(sim == scores, colsf, float(split)),
                  axis=1, keepdims=True)
    scores_r = jnp.transpose(scores)

    rank_cond = (scores_r > scores) | ((scores_r == scores) & (colsf < rowsf))
    rank = jax.lax.dot_general(
        rank_cond.astype(f32), ones_col, (((1,), (0,)), ((), ())),
        preferred_element_type=f32)
    merged_c = rank < r

    stride = 2.0 * split
    z = jnp.where(merged_c, ind * stride + rank, rank - _BIG)
    z_r = jnp.transpose(z)
    merged_rb = z_r >= 0.0
    ind_rr = jnp.floor(z_r * (1.0 / stride))
    rank_r = jnp.where(merged_rb, z_r - stride * ind_rr, z_r + _BIG)

    dz = z - z_r
    later_same = (dz > 0.0) & (dz < float(split))
    c_r = jnp.sum(later_same.astype(f32), axis=0, keepdims=True)
    w_r = _pow2_neg((c_r + 1.0).astype(jnp.int32))

    kept_above = (rowsf < colsf) & jnp.logical_not(merged_c)
    kr_r = jnp.sum(kept_above.astype(f32), axis=0, keepdims=True)

    t_r = jnp.where(merged_rb, float(kept) + ind_rr, kr_r)
    v_r = jnp.where(merged_rb, w_r, 1.0) * ns_r
    q_iota = jax.lax.broadcasted_iota(
        jnp.int32, (rows_c, split), 0).astype(f32)
    E = (q_iota == t_r).astype(f32)
    C = E * v_r
    out2 = jax.lax.dot_general(
        C, sn, (((1,), (0,)), ((), ())), preferred_element_type=f32)
    m = jax.lax.dot_general(
        E, merged_c.astype(f32), (((1,), (0,)), ((), ())),
        preferred_element_type=f32)[kept:kept + split]

    dst_m = dn * (nd_c * _pow2_neg(m.astype(jnp.int32))) \
        + out2[kept:kept + split]
    merged_ref[k] = jnp.concatenate(
        [cls_ref[k], out2[:kept], dst_m], axis=0)

    sel = (rank_r == rowsf).astype(f32)
    midx_c = jnp.sum(sel * colsf, axis=1, keepdims=True)
    midx_ref[k] = midx_c[:r].astype(jnp.int32)
    ind_ref[k] = ind.astype(jnp.int32)


def kernel(tokens, cls_token):
    B, N, D = tokens.shape
    split = N // 2
    r = int(N * _MERGE_RATIO)
    r = min(r, N - _MIN_TOKENS)
    kept = split - r
    n_out = 1 + kept + split
    rows_c = ((kept + split + 7) // 8) * 8

    src, dst = tokens[:, :split], tokens[:, split:]
    ns = jnp.linalg.norm(src, axis=-1, keepdims=True)
    nd = jnp.linalg.norm(dst, axis=-1, keepdims=True)
    ns = jnp.maximum(ns, 1e-12)
    nd = jnp.maximum(nd, 1e-12)
    src_n = src / ns
    dst_n = dst / nd

    bb = 4 if B % 4 == 0 else 1
    kfn = functools.partial(_merge_kernel, split=split, r=r, rows_c=rows_c,
                            bb=bb)
    merged, midx, ind = pl.pallas_call(
        kfn,
        grid=(B // bb,),
        in_specs=[
            pl.BlockSpec((bb, split, D), lambda b: (b, 0, 0)),
            pl.BlockSpec((bb, split, D), lambda b: (b, 0, 0)),
            pl.BlockSpec((bb, 1, split), lambda b: (b, 0, 0)),
            pl.BlockSpec((bb, 1, split), lambda b: (b, 0, 0)),
            pl.BlockSpec((bb, 1, D), lambda b: (b, 0, 0)),
        ],
        out_specs=[
            pl.BlockSpec((bb, n_out, D), lambda b: (b, 0, 0)),
            pl.BlockSpec((bb, r, 1), lambda b: (b, 0, 0)),
            pl.BlockSpec((bb, split, 1), lambda b: (b, 0, 0)),
        ],
        out_shape=[
            jax.ShapeDtypeStruct((B, n_out, D), tokens.dtype),
            jax.ShapeDtypeStruct((B, r, 1), jnp.int32),
            jax.ShapeDtypeStruct((B, split, 1), jnp.int32),
        ],
    )(src_n, dst_n, ns.reshape(B, 1, split), nd.reshape(B, 1, split),
      cls_token)
    return merged, midx.reshape(B, r), ind.reshape(B, split)


# speedup vs baseline: 18.8713x; 1.1988x over previous
import functools

import jax
import jax.numpy as jnp
from jax.experimental import pallas as pl

_MERGE_RATIO = 0.3
_MIN_TOKENS = 4
_BIG = 8388608.0


def _pow2_neg(k):
    e = jnp.maximum(127 - k, 0) << 23
    return jax.lax.bitcast_convert_type(e.astype(jnp.int32), jnp.float32)


def _merge_kernel(tok_ref, ssqs_ref, ssqd_ref, cls_ref,
                  merged_ref, midx_ref, ind_ref, *, split, r, rows_c, bb):
    for k in range(bb):
        _merge_one(tok_ref, ssqs_ref, ssqd_ref, cls_ref,
                   merged_ref, midx_ref, ind_ref, k,
                   split=split, r=r, rows_c=rows_c)


def _merge_one(tok_ref, ssqs_ref, ssqd_ref, cls_ref,
               merged_ref, midx_ref, ind_ref, k, *, split, r, rows_c):
    f32 = jnp.float32
    kept = split - r
    src = tok_ref[k, :split, :]
    dst = tok_ref[k, split:, :]
    ns_r = jnp.maximum(jnp.sqrt(ssqs_ref[k]), 1e-12)
    nd_r = jnp.maximum(jnp.sqrt(ssqd_ref[k]), 1e-12)
    sn = src / jnp.transpose(ns_r)
    dn = dst / jnp.transpose(nd_r)

    sim = jax.lax.dot_general(
        sn, dn, (((1,), (1,)), ((), ())), preferred_element_type=f32)

    rowsf = jax.lax.broadcasted_iota(jnp.int32, (split, split), 0).astype(f32)
    colsf = jax.lax.broadcasted_iota(jnp.int32, (split, split), 1).astype(f32)
    ones_col = jnp.ones((split, 1), f32)

    scores = jnp.max(sim, axis=1, keepdims=True)
    ind = jnp.min(jnp.where(sim == scores, colsf, float(split)),
                  axis=1, keepdims=True)
    scores_r = jnp.transpose(scores)

    rank_cond = (scores_r > scores) | ((scores_r == scores) & (colsf < rowsf))
    rank = jax.lax.dot_general(
        rank_cond.astype(f32), ones_col, (((1,), (0,)), ((), ())),
        preferred_element_type=f32)
    merged_c = rank < r

    stride = 2.0 * split
    z = jnp.where(merged_c, ind * stride + rank, rank - _BIG)
    z_r = jnp.transpose(z)
    merged_rb = z_r >= 0.0
    ind_rr = jnp.floor(z_r * (1.0 / stride))
    rank_r = jnp.where(merged_rb, z_r - stride * ind_rr, z_r + _BIG)

    dz = z - z_r
    later_same = (dz > 0.0) & (dz < float(split))
    c_r = jnp.sum(later_same.astype(f32), axis=0, keepdims=True)
    w_r = _pow2_neg((c_r + 1.0).astype(jnp.int32))

    kept_above = (rowsf < colsf) & jnp.logical_not(merged_c)
    kr_r = jnp.sum(kept_above.astype(f32), axis=0, keepdims=True)

    t_r = jnp.where(merged_rb, float(kept) + ind_rr, kr_r)
    v_r = jnp.where(merged_rb, w_r, 1.0)
    q_iota = jax.lax.broadcasted_iota(
        jnp.int32, (rows_c, split), 0).astype(f32)
    E = (q_iota == t_r).astype(f32)
    C = E * v_r
    out2 = jax.lax.dot_general(
        C, src, (((1,), (0,)), ((), ())), preferred_element_type=f32)
    m = jax.lax.dot_general(
        E, merged_c.astype(f32), (((1,), (0,)), ((), ())),
        preferred_element_type=f32)[kept:kept + split]

    dst_m = dst * _pow2_neg(m.astype(jnp.int32)) + out2[kept:kept + split]
    merged_ref[k] = jnp.concatenate(
        [cls_ref[k], out2[:kept], dst_m], axis=0)

    sel = (rank_r == rowsf).astype(f32)
    midx_c = jnp.sum(sel * colsf, axis=1, keepdims=True)
    midx_ref[k] = midx_c[:r].astype(jnp.int32)
    ind_ref[k] = ind.astype(jnp.int32)


def kernel(tokens, cls_token):
    B, N, D = tokens.shape
    split = N // 2
    r = int(N * _MERGE_RATIO)
    r = min(r, N - _MIN_TOKENS)
    kept = split - r
    n_out = 1 + kept + split
    rows_c = ((kept + split + 7) // 8) * 8

    ssq = jnp.sum(tokens * tokens, axis=-1)
    ssq_s = ssq[:, :split].reshape(B, 1, split)
    ssq_d = ssq[:, split:].reshape(B, 1, split)

    bb = 4 if B % 4 == 0 else 1
    kfn = functools.partial(_merge_kernel, split=split, r=r, rows_c=rows_c,
                            bb=bb)
    merged, midx, ind = pl.pallas_call(
        kfn,
        grid=(B // bb,),
        in_specs=[
            pl.BlockSpec((bb, N, D), lambda b: (b, 0, 0)),
            pl.BlockSpec((bb, 1, split), lambda b: (b, 0, 0)),
            pl.BlockSpec((bb, 1, split), lambda b: (b, 0, 0)),
            pl.BlockSpec((bb, 1, D), lambda b: (b, 0, 0)),
        ],
        out_specs=[
            pl.BlockSpec((bb, n_out, D), lambda b: (b, 0, 0)),
            pl.BlockSpec((bb, r, 1), lambda b: (b, 0, 0)),
            pl.BlockSpec((bb, split, 1), lambda b: (b, 0, 0)),
        ],
        out_shape=[
            jax.ShapeDtypeStruct((B, n_out, D), tokens.dtype),
            jax.ShapeDtypeStruct((B, r, 1), jnp.int32),
            jax.ShapeDtypeStruct((B, split, 1), jnp.int32),
        ],
    )(tokens, ssq_s, ssq_d, cls_token)
    return merged, midx.reshape(B, r), ind.reshape(B, split)
